# baseline (speedup 1.0000x reference)
import functools

import jax
import jax.numpy as jnp
from jax import lax
from jax.experimental import pallas as pl
from jax.experimental.pallas import tpu as pltpu
from jax.experimental.pallas import tpu_sc as plsc

_N_FEATURES = 256
_FEATURE_INDEX = 3
_LANES = 16


def _build_sc_call(batch):
    info = plsc.get_sparse_core_info()
    num_cores, num_subcores = 1, 8
    num_workers = num_cores * num_subcores
    per_worker = batch // num_workers
    assert per_worker % _LANES == 0 and per_worker % 8 == 0

    mesh = plsc.VectorSubcoreMesh(
        core_axis_name="c",
        subcore_axis_name="s",
        num_cores=num_cores,
        num_subcores=num_subcores,
    )

    @functools.partial(
        pl.kernel,
        mesh=mesh,
        out_type=jax.ShapeDtypeStruct((batch,), jnp.float32),
        scratch_types=[
            pltpu.VMEM((per_worker,), jnp.float32),
            pltpu.VMEM((_LANES,), jnp.float32),
            pltpu.VMEM((_LANES,), jnp.float32),
            pltpu.SemaphoreType.DMA,
        ],
    )
    def sc_affine(x_hbm, scale_hbm, min_hbm, out_hbm, x_v, sc_v, mn_v, sem):
        wid = lax.axis_index("s") * num_cores + lax.axis_index("c")
        base = wid * per_worker
        window = (_FEATURE_INDEX // _LANES) * _LANES
        cp_x = pltpu.async_copy(x_hbm.at[pl.ds(base, per_worker)], x_v, sem)
        cp_s = pltpu.async_copy(scale_hbm.at[pl.ds(window, _LANES)], sc_v, sem)
        cp_m = pltpu.async_copy(min_hbm.at[pl.ds(window, _LANES)], mn_v, sem)
        cp_x.wait()
        cp_s.wait()
        cp_m.wait()
        s = jnp.full((_LANES,), sc_v[...][_FEATURE_INDEX - window])
        m = jnp.full((_LANES,), mn_v[...][_FEATURE_INDEX - window])
        for i in range(per_worker // _LANES):
            sl = pl.ds(i * _LANES, _LANES)
            x_v[sl] = x_v[sl] * s + m
        pltpu.sync_copy(x_v, out_hbm.at[pl.ds(base, per_worker)])

    return sc_affine


def kernel(inputs, scaler_scale, scaler_min):
    batch = inputs.shape[0]
    x = inputs.reshape(batch)
    return _build_sc_call(batch)(x, scaler_scale, scaler_min)


# speedup vs baseline: 1.0195x; 1.0195x over previous
import functools

import jax
import jax.numpy as jnp
from jax import lax
from jax.experimental import pallas as pl
from jax.experimental.pallas import tpu as pltpu
from jax.experimental.pallas import tpu_sc as plsc

_N_FEATURES = 256
_FEATURE_INDEX = 3
_LANES = 16


def _build_sc_call(batch):
    info = plsc.get_sparse_core_info()
    num_cores, num_subcores = 1, info.num_subcores
    num_workers = num_cores * num_subcores
    per_worker = batch // num_workers
    assert per_worker % _LANES == 0 and per_worker % 8 == 0

    mesh = plsc.VectorSubcoreMesh(
        core_axis_name="c",
        subcore_axis_name="s",
        num_cores=num_cores,
        num_subcores=num_subcores,
    )

    @functools.partial(
        pl.kernel,
        mesh=mesh,
        out_type=jax.ShapeDtypeStruct((batch,), jnp.float32),
        scratch_types=[
            pltpu.VMEM((per_worker,), jnp.float32),
            pltpu.VMEM((_LANES,), jnp.float32),
            pltpu.VMEM((_LANES,), jnp.float32),
            pltpu.SemaphoreType.DMA,
        ],
    )
    def sc_affine(x_hbm, scale_hbm, min_hbm, out_hbm, x_v, sc_v, mn_v, sem):
        wid = lax.axis_index("s") * num_cores + lax.axis_index("c")
        base = wid * per_worker
        window = (_FEATURE_INDEX // _LANES) * _LANES
        cp_x = pltpu.async_copy(x_hbm.at[pl.ds(base, per_worker)], x_v, sem)
        cp_s = pltpu.async_copy(scale_hbm.at[pl.ds(window, _LANES)], sc_v, sem)
        cp_m = pltpu.async_copy(min_hbm.at[pl.ds(window, _LANES)], mn_v, sem)
        cp_x.wait()
        cp_s.wait()
        cp_m.wait()
        s = jnp.full((_LANES,), sc_v[...][_FEATURE_INDEX - window])
        m = jnp.full((_LANES,), mn_v[...][_FEATURE_INDEX - window])
        for i in range(per_worker // _LANES):
            sl = pl.ds(i * _LANES, _LANES)
            x_v[sl] = x_v[sl] * s + m
        pltpu.sync_copy(x_v, out_hbm.at[pl.ds(base, per_worker)])

    return sc_affine


def kernel(inputs, scaler_scale, scaler_min):
    batch = inputs.shape[0]
    x = inputs.reshape(batch)
    return _build_sc_call(batch)(x, scaler_scale, scaler_min)


# speedup vs baseline: 1.0245x; 1.0049x over previous
import functools

import jax
import jax.numpy as jnp
from jax import lax
from jax.experimental import pallas as pl
from jax.experimental.pallas import tpu as pltpu
from jax.experimental.pallas import tpu_sc as plsc

_FEATURE_INDEX = 3
_LANES = 16


def _build_sc_call(batch):
    info = plsc.get_sparse_core_info()
    num_cores, num_subcores = 1, info.num_subcores
    num_workers = num_cores * num_subcores
    per_worker = batch // num_workers
    assert per_worker % _LANES == 0 and per_worker % 8 == 0

    mesh = plsc.VectorSubcoreMesh(
        core_axis_name="c",
        subcore_axis_name="s",
        num_cores=num_cores,
        num_subcores=num_subcores,
    )

    @functools.partial(
        pl.kernel,
        mesh=mesh,
        out_type=jax.ShapeDtypeStruct((batch,), jnp.float32),
        scratch_types=[
            pltpu.VMEM((per_worker,), jnp.float32),
            pltpu.VMEM((_LANES,), jnp.float32),
            pltpu.VMEM((_LANES,), jnp.float32),
            pltpu.SemaphoreType.DMA,
        ],
    )
    def sc_affine(x_hbm, scale_hbm, min_hbm, out_hbm, x_v, sc_v, mn_v, sem):
        wid = lax.axis_index("s") * num_cores + lax.axis_index("c")
        base = wid * per_worker
        window = (_FEATURE_INDEX // _LANES) * _LANES
        cp_x = pltpu.async_copy(x_hbm.at[pl.ds(base, per_worker)], x_v, sem)
        cp_s = pltpu.async_copy(scale_hbm.at[pl.ds(window, _LANES)], sc_v, sem)
        cp_m = pltpu.async_copy(min_hbm.at[pl.ds(window, _LANES)], mn_v, sem)
        cp_x.wait()
        cp_s.wait()
        cp_m.wait()
        s = jnp.full((_LANES,), sc_v[...][_FEATURE_INDEX - window])
        m = jnp.full((_LANES,), mn_v[...][_FEATURE_INDEX - window])
        for i in range(per_worker // _LANES):
            sl = pl.ds(i * _LANES, _LANES)
            x_v[sl] = x_v[sl] * s + m
        pltpu.sync_copy(x_v, out_hbm.at[pl.ds(base, per_worker)])

    return sc_affine


def kernel(inputs, scaler_scale, scaler_min):
    batch = inputs.shape[0]
    x = inputs.reshape(batch)
    return _build_sc_call(batch)(x, scaler_scale, scaler_min)
